# baseline (speedup 1.0000x reference)
import functools

import jax
import jax.numpy as jnp
from jax import lax
from jax.experimental import pallas as pl
from jax.experimental.pallas import tpu as pltpu
from jax.experimental.pallas import tpu_sc as plsc

NC = 2
NS = 16
NW = NC * NS
L = 16
C = 128

_Q = ((1, 1, 1), (1, 1, 0), (1, 0, 1), (0, 1, 1),
      (1, 0, 0), (0, 1, 0), (0, 0, 1), (0, 0, 0))


TS = 512


def _sc_body(vpw, n_chunks, k_pad, tail_pw, tail_rem, d,
             table, feats2d, p3, out,
             idxbufs, rowbufs, pbufs, outbufs, tailbufs,
             gsems, rsems, wsems):
    wid = lax.axis_index("s") * NC + lax.axis_index("c")
    wbase = wid * vpw

    n_tsteps = tail_pw // TS
    t_left = tail_pw - n_tsteps * TS
    tbase = pl.multiple_of(k_pad + wid * tail_pw, 8)

    def tail_read(step, b):
        off = pl.multiple_of(tbase + step * TS, 8)
        pltpu.async_copy(table.at[pl.ds(off, TS)], tailbufs[b], rsems[b])

    def tail_drain_read(b):
        pltpu.make_async_copy(table.at[pl.ds(0, TS)], tailbufs[b],
                              rsems[b]).wait()

    def tail_write(step, b):
        off = pl.multiple_of(tbase + step * TS, 8)
        pltpu.async_copy(tailbufs[b], out.at[pl.ds(off, TS)], wsems[b])

    def tail_drain_write(b):
        pltpu.make_async_copy(tailbufs[b], out.at[pl.ds(0, TS)],
                              wsems[b]).wait()

    def stage(chunk, b):
        voff = pl.multiple_of(wbase + chunk * C, C)
        pltpu.sync_copy(feats2d.at[pl.ds(pl.multiple_of(voff // 16, 8), 8)],
                        idxbufs[b])
        pltpu.sync_copy(p3.at[:, pl.ds(voff, C)], pbufs[b])
        for r in range(8):
            pltpu.async_copy(table.at[idxbufs[b].at[r]],
                             rowbufs[b].at[pl.ds(r * 128, 128)], gsems[b])

    def drain(b):
        pltpu.make_async_copy(table.at[pl.ds(0, C * 8)], rowbufs[b],
                              gsems[b]).wait()

    def compute(chunk, b):
        rows = rowbufs[b]
        pbuf = pbufs[b]
        outbuf = outbufs[b]

        def group_body(g, carry2):
            px = pbuf[0, pl.ds(g * L, L)]
            py = pbuf[1, pl.ds(g * L, L)]
            pz = pbuf[2, pl.ds(g * L, L)]
            one = jnp.float32(1.0)
            tx = (px, one - px)
            ty = (py, one - py)
            tz = (pz, one - pz)
            wvecs = []
            xy = {}
            for j in range(8):
                qx, qy, qz = _Q[j]
                if (qx, qy) not in xy:
                    xy[(qx, qy)] = tx[1 - qx] * ty[1 - qy]
                wvecs.append(xy[(qx, qy)] * tz[1 - qz])
            for i in range(16):
                rowb = g * 128 + i * 8
                acc_lo = None
                acc_hi = None
                for j in range(8):
                    wsp = jnp.broadcast_to(wvecs[j][i], (L,))
                    rlo = rows[rowb + j, pl.ds(0, L)]
                    rhi = rows[rowb + j, pl.ds(L, L)]
                    if acc_lo is None:
                        acc_lo = wsp * rlo
                        acc_hi = wsp * rhi
                    else:
                        acc_lo = acc_lo + wsp * rlo
                        acc_hi = acc_hi + wsp * rhi
                outbuf[g * L + i, pl.ds(0, L)] = acc_lo
                outbuf[g * L + i, pl.ds(L, L)] = acc_hi
            return carry2

        lax.fori_loop(0, C // L, group_body, 0, unroll=False)
        voff = pl.multiple_of(wbase + chunk * C, C)
        pltpu.sync_copy(outbuf, out.at[pl.ds(voff, C)])

    assert n_tsteps + 1 <= n_chunks
    stage(0, 0)
    tail_read(0, 0)

    def pair_body(c2, carry):
        for b in range(2):
            chunk = c2 * 2 + b
            drain(b)

            @pl.when(chunk + 1 < n_chunks)
            def _():
                stage(chunk + 1, 1 - b)

            s = chunk

            @pl.when(s < n_tsteps)
            def _():
                tail_drain_read(b)
                tail_write(s, b)

            @pl.when(s + 1 < n_tsteps)
            def _():
                @pl.when(s >= 1)
                def _():
                    tail_drain_write(1 - b)
                tail_read(s + 1, 1 - b)

            compute(chunk, b)
        return carry

    assert n_chunks % 2 == 0
    lax.fori_loop(0, n_chunks // 2, pair_body, 0, unroll=False)
    tail_drain_write((n_tsteps - 2) % 2)
    tail_drain_write((n_tsteps - 1) % 2)
    if t_left:
        off = pl.multiple_of(tbase + n_tsteps * TS, 8)
        pltpu.sync_copy(table.at[pl.ds(off, t_left)],
                        tailbufs[0].at[pl.ds(0, t_left)])
        pltpu.sync_copy(tailbufs[0].at[pl.ds(0, t_left)],
                        out.at[pl.ds(off, t_left)])
    if tail_rem:
        rem0 = k_pad + NW * tail_pw

        @pl.when(wid == 0)
        def _():
            pltpu.sync_copy(table.at[pl.ds(rem0, tail_rem)],
                            tailbufs[1].at[pl.ds(0, tail_rem)])
            pltpu.sync_copy(tailbufs[1].at[pl.ds(0, tail_rem)],
                            out.at[pl.ds(rem0, tail_rem)])


def kernel(values_weight, p, feats, write_idx):
    m, d = values_weight.shape
    k = p.shape[0]
    del write_idx

    vpw = -(-k // (NW * 2 * C)) * 2 * C
    k_pad = vpw * NW
    n_chunks = vpw // C
    tail = m - k_pad
    assert d == 2 * L
    tail_pw = tail // NW // 8 * 8
    tail_rem = tail - NW * tail_pw

    pad_rows = jnp.arange(k, k_pad, dtype=jnp.int32)
    feats_pad = jnp.concatenate(
        [feats, jnp.broadcast_to(pad_rows[:, None], (k_pad - k, 8))], axis=0)
    feats2d = feats_pad.reshape(k_pad * 8 // 128, 128)
    p2 = p.reshape(k, 3)
    p3 = jnp.concatenate(
        [p2, jnp.full((k_pad - k, 3), 0.5, jnp.float32)], axis=0).T

    body = functools.partial(_sc_body, vpw, n_chunks, k_pad, tail_pw,
                             tail_rem, d)
    f = pl.kernel(
        body,
        out_type=jax.ShapeDtypeStruct((m, d), jnp.float32),
        mesh=plsc.VectorSubcoreMesh(core_axis_name="c", subcore_axis_name="s"),
        scratch_types=[
            [pltpu.VMEM((8, 128), jnp.int32)] * 2,
            [pltpu.VMEM((C * 8, d), jnp.float32)] * 2,
            [pltpu.VMEM((3, C), jnp.float32)] * 2,
            [pltpu.VMEM((C, d), jnp.float32)] * 2,
            [pltpu.VMEM((TS, d), jnp.float32)] * 2,
            [pltpu.SemaphoreType.DMA] * 2,
            [pltpu.SemaphoreType.DMA] * 2,
            [pltpu.SemaphoreType.DMA] * 2,
        ],
        compiler_params=pltpu.CompilerParams(use_tc_tiling_on_sc=False),
    )
    return f(values_weight, feats2d, p3)


# speedup vs baseline: 1.3382x; 1.3382x over previous
import functools

import jax
import jax.numpy as jnp
from jax import lax
from jax.experimental import pallas as pl
from jax.experimental.pallas import tpu as pltpu
from jax.experimental.pallas import tpu_sc as plsc

NC = 2
NS = 16
NW = NC * NS
L = 16
C = 128
BC = 8192

_Q = ((1, 1, 1), (1, 1, 0), (1, 0, 1), (0, 1, 1),
      (1, 0, 0), (0, 1, 0), (0, 0, 1), (0, 0, 0))


def _sc_body(vpw, n_chunks, d,
             table, feats2d, p3, out,
             idxbufs, rowbufs, pbufs, outbufs, gsems):
    wid = lax.axis_index("s") * NC + lax.axis_index("c")
    wbase = wid * vpw

    def stage(chunk, b):
        voff = pl.multiple_of(wbase + chunk * C, C)
        pltpu.sync_copy(feats2d.at[pl.ds(pl.multiple_of(voff // 16, 8), 8)],
                        idxbufs[b])
        pltpu.sync_copy(p3.at[:, pl.ds(voff, C)], pbufs[b])
        for r in range(8):
            pltpu.async_copy(table.at[idxbufs[b].at[r]],
                             rowbufs[b].at[pl.ds(r * 128, 128)], gsems[b])

    def drain(b):
        pltpu.make_async_copy(table.at[pl.ds(0, C * 8)], rowbufs[b],
                              gsems[b]).wait()

    def compute(chunk, b):
        rows = rowbufs[b]
        pbuf = pbufs[b]
        outbuf = outbufs[b]

        def group_body(g, carry2):
            px = pbuf[0, pl.ds(g * L, L)]
            py = pbuf[1, pl.ds(g * L, L)]
            pz = pbuf[2, pl.ds(g * L, L)]
            one = jnp.float32(1.0)
            tx = (px, one - px)
            ty = (py, one - py)
            tz = (pz, one - pz)
            wvecs = []
            xy = {}
            for j in range(8):
                qx, qy, qz = _Q[j]
                if (qx, qy) not in xy:
                    xy[(qx, qy)] = tx[1 - qx] * ty[1 - qy]
                wvecs.append(xy[(qx, qy)] * tz[1 - qz])
            for i in range(16):
                rowb = g * 128 + i * 8
                acc_lo = None
                acc_hi = None
                for j in range(8):
                    wsp = jnp.broadcast_to(wvecs[j][i], (L,))
                    rlo = rows[rowb + j, pl.ds(0, L)]
                    rhi = rows[rowb + j, pl.ds(L, L)]
                    if acc_lo is None:
                        acc_lo = wsp * rlo
                        acc_hi = wsp * rhi
                    else:
                        acc_lo = acc_lo + wsp * rlo
                        acc_hi = acc_hi + wsp * rhi
                outbuf[g * L + i, pl.ds(0, L)] = acc_lo
                outbuf[g * L + i, pl.ds(L, L)] = acc_hi
            return carry2

        lax.fori_loop(0, C // L, group_body, 0, unroll=False)
        voff = pl.multiple_of(wbase + chunk * C, C)
        pltpu.sync_copy(outbuf, out.at[pl.ds(voff, C)])

    stage(0, 0)

    def pair_body(c2, carry):
        for b in range(2):
            chunk = c2 * 2 + b
            drain(b)

            @pl.when(chunk + 1 < n_chunks)
            def _():
                stage(chunk + 1, 1 - b)

            compute(chunk, b)
        return carry

    assert n_chunks % 2 == 0
    lax.fori_loop(0, n_chunks // 2, pair_body, 0, unroll=False)


def _assemble_body(nhead, vw_t, nv_t, out_t):
    i = pl.program_id(0)

    @pl.when(i < nhead)
    def _():
        out_t[...] = nv_t[...]

    @pl.when(i >= nhead)
    def _():
        out_t[...] = vw_t[...]


def kernel(values_weight, p, feats, write_idx):
    m, d = values_weight.shape
    k = p.shape[0]
    del write_idx

    vpw = -(-k // (NW * 2 * C)) * 2 * C
    k_pad = vpw * NW
    n_chunks = vpw // C
    assert d == 2 * L and k_pad % BC == 0

    pad_rows = jnp.arange(k, k_pad, dtype=jnp.int32)
    feats_pad = jnp.concatenate(
        [feats, jnp.broadcast_to(pad_rows[:, None], (k_pad - k, 8))], axis=0)
    feats2d = feats_pad.reshape(k_pad * 8 // 128, 128)
    p2 = p.reshape(k, 3)
    p3 = jnp.concatenate(
        [p2, jnp.full((k_pad - k, 3), 0.5, jnp.float32)], axis=0).T

    sc = pl.kernel(
        functools.partial(_sc_body, vpw, n_chunks, d),
        out_type=jax.ShapeDtypeStruct((k_pad, d), jnp.float32),
        mesh=plsc.VectorSubcoreMesh(core_axis_name="c", subcore_axis_name="s"),
        scratch_types=[
            [pltpu.VMEM((8, 128), jnp.int32)] * 2,
            [pltpu.VMEM((C * 8, d), jnp.float32)] * 2,
            [pltpu.VMEM((3, C), jnp.float32)] * 2,
            [pltpu.VMEM((C, d), jnp.float32)] * 2,
            [pltpu.SemaphoreType.DMA] * 2,
        ],
        compiler_params=pltpu.CompilerParams(use_tc_tiling_on_sc=False),
    )
    new_vals = sc(values_weight, feats2d, p3)

    vw_t = jnp.swapaxes(values_weight, 0, 1)
    nv_t = jnp.swapaxes(new_vals, 0, 1)
    nhead = k_pad // BC
    out_t = pl.pallas_call(
        functools.partial(_assemble_body, nhead),
        grid=(-(-m // BC),),
        in_specs=[
            pl.BlockSpec((d, BC), lambda i: (0, i)),
            pl.BlockSpec((d, BC), lambda i, _n=nhead: (0, jnp.minimum(i, _n - 1))),
        ],
        out_specs=pl.BlockSpec((d, BC), lambda i: (0, i)),
        out_shape=jax.ShapeDtypeStruct((d, m), jnp.float32),
    )(vw_t, nv_t)
    return jnp.swapaxes(out_t, 0, 1)


# speedup vs baseline: 1.4218x; 1.0624x over previous
import functools

import jax
import jax.numpy as jnp
from jax import lax
from jax.experimental import pallas as pl
from jax.experimental.pallas import tpu as pltpu
from jax.experimental.pallas import tpu_sc as plsc

NC = 2
NS = 16
NW = NC * NS
L = 16
C = 128
BC = 8192

_Q = ((1, 1, 1), (1, 1, 0), (1, 0, 1), (0, 1, 1),
      (1, 0, 0), (0, 1, 0), (0, 0, 1), (0, 0, 0))


def _sc_body(vpw, n_chunks, d,
             table, feats_t, p3, out,
             idxbufs, rowbufs, pbufs, outbufs, gsems):
    wid = lax.axis_index("s") * NC + lax.axis_index("c")
    wbase = wid * vpw

    def stage(chunk, b):
        voff = pl.multiple_of(wbase + chunk * C, C)
        pltpu.sync_copy(feats_t.at[:, pl.ds(voff, C)], idxbufs[b])
        pltpu.sync_copy(p3.at[:, pl.ds(voff, C)], pbufs[b])
        for r in range(8):
            pltpu.async_copy(table.at[idxbufs[b].at[r]],
                             rowbufs[b].at[pl.ds(r * 128, 128)], gsems[b])

    def drain(b):
        pltpu.make_async_copy(table.at[pl.ds(0, C * 8)], rowbufs[b],
                              gsems[b]).wait()

    def compute(chunk, b):
        rows = rowbufs[b]
        pbuf = pbufs[b]
        outbuf = outbufs[b]

        def group_body(g, carry2):
            px = pbuf[0, pl.ds(g * L, L)]
            py = pbuf[1, pl.ds(g * L, L)]
            pz = pbuf[2, pl.ds(g * L, L)]
            one = jnp.float32(1.0)
            tx = (px, one - px)
            ty = (py, one - py)
            tz = (pz, one - pz)
            wvecs = []
            xy = {}
            for j in range(8):
                qx, qy, qz = _Q[j]
                if (qx, qy) not in xy:
                    xy[(qx, qy)] = tx[1 - qx] * ty[1 - qy]
                wvecs.append(xy[(qx, qy)] * tz[1 - qz])
            for i in range(16):
                rowb = g * L + i
                acc_lo = None
                acc_hi = None
                for j in range(8):
                    wsp = jnp.broadcast_to(wvecs[j][i], (L,))
                    rlo = rows[rowb + j * C, pl.ds(0, L)]
                    rhi = rows[rowb + j * C, pl.ds(L, L)]
                    if acc_lo is None:
                        acc_lo = wsp * rlo
                        acc_hi = wsp * rhi
                    else:
                        acc_lo = acc_lo + wsp * rlo
                        acc_hi = acc_hi + wsp * rhi
                outbuf[g * L + i, pl.ds(0, L)] = acc_lo
                outbuf[g * L + i, pl.ds(L, L)] = acc_hi
            return carry2

        lax.fori_loop(0, C // L, group_body, 0, unroll=False)
        voff = pl.multiple_of(wbase + chunk * C, C)
        pltpu.sync_copy(outbuf, out.at[pl.ds(voff, C)])

    stage(0, 0)

    def pair_body(c2, carry):
        for b in range(2):
            chunk = c2 * 2 + b
            drain(b)

            @pl.when(chunk + 1 < n_chunks)
            def _():
                stage(chunk + 1, 1 - b)

            compute(chunk, b)
        return carry

    assert n_chunks % 2 == 0
    lax.fori_loop(0, n_chunks // 2, pair_body, 0, unroll=False)


def _copy_body(vw_t, out_t):
    out_t[...] = vw_t[...]


def _insert_body(base_t, nv_t, out_t):
    del base_t
    out_t[...] = nv_t[...]


def kernel(values_weight, p, feats, write_idx):
    m, d = values_weight.shape
    k = p.shape[0]
    del write_idx

    vpw = -(-k // (NW * 2 * C)) * 2 * C
    k_pad = vpw * NW
    n_chunks = vpw // C
    assert d == 2 * L and k_pad % BC == 0

    pad_rows = jnp.arange(k, k_pad, dtype=jnp.int32)
    feats_t = jnp.concatenate(
        [jnp.swapaxes(feats, 0, 1),
         jnp.broadcast_to(pad_rows[None, :], (8, k_pad - k))], axis=1)
    p2 = p.reshape(k, 3)
    p3 = jnp.concatenate(
        [p2, jnp.full((k_pad - k, 3), 0.5, jnp.float32)], axis=0).T

    sc = pl.kernel(
        functools.partial(_sc_body, vpw, n_chunks, d),
        out_type=jax.ShapeDtypeStruct((k_pad, d), jnp.float32),
        mesh=plsc.VectorSubcoreMesh(core_axis_name="c", subcore_axis_name="s"),
        scratch_types=[
            [pltpu.VMEM((8, 128), jnp.int32)] * 2,
            [pltpu.VMEM((C * 8, d), jnp.float32)] * 2,
            [pltpu.VMEM((3, C), jnp.float32)] * 2,
            [pltpu.VMEM((C, d), jnp.float32)] * 2,
            [pltpu.SemaphoreType.DMA] * 2,
        ],
        compiler_params=pltpu.CompilerParams(use_tc_tiling_on_sc=False),
    )
    new_vals = sc(values_weight, feats_t, p3)

    vw_t = jnp.swapaxes(values_weight, 0, 1)
    nv_t = jnp.swapaxes(new_vals, 0, 1)
    nhead = k_pad // BC
    base_t = pl.pallas_call(
        _copy_body,
        grid=(-(-m // BC),),
        in_specs=[pl.BlockSpec((d, BC), lambda i: (0, i))],
        out_specs=pl.BlockSpec((d, BC), lambda i: (0, i)),
        out_shape=jax.ShapeDtypeStruct((d, m), jnp.float32),
    )(vw_t)
    out_t = pl.pallas_call(
        _insert_body,
        grid=(nhead,),
        in_specs=[
            pl.BlockSpec(memory_space=pl.ANY),
            pl.BlockSpec((d, BC), lambda i: (0, i)),
        ],
        out_specs=pl.BlockSpec((d, BC), lambda i: (0, i)),
        out_shape=jax.ShapeDtypeStruct((d, m), jnp.float32),
        input_output_aliases={0: 0},
    )(base_t, nv_t)
    return jnp.swapaxes(out_t, 0, 1)
